# baseline (speedup 1.0000x reference)
import functools
import jax
import jax.numpy as jnp
from jax import lax
from jax.experimental import pallas as pl
from jax.experimental.pallas import tpu as pltpu

N = 10000
E = 160000
D_IN = 256
D_H = 512
D_E = 16

E_PAD = 163840
BE = 2048


def _edge_mlp_body(a_ref, lw_ref, lb_ref, o_ref):
    a = a_ref[...]
    lw = lw_ref[...]
    lb = lb_ref[...]
    o_ref[...] = jnp.dot(a, lw, preferred_element_type=jnp.float32) + lb


def _edge_mlp(edge_attr_pad, lw, lb, S):
    nblk = E_PAD // BE
    return pl.pallas_call(
        _edge_mlp_body,
        grid=(S, nblk),
        in_specs=[
            pl.BlockSpec((BE, D_E), lambda p, i: (i, 0)),
            pl.BlockSpec((D_E, 128), lambda p, i: (0, p)),
            pl.BlockSpec((1, 128), lambda p, i: (0, p)),
        ],
        out_specs=pl.BlockSpec((BE, 128), lambda p, i: (p * nblk + i, 0)),
        out_shape=jax.ShapeDtypeStruct((S * E_PAD, 128), jnp.float32),
    )(edge_attr_pad, lw, lb.reshape(1, S * 128))


BN = 1000


def _mlp0_body(x_ref, agg_ref, w1_ref, b1_ref, w2_ref, b2_ref,
               skw_ref, skb_ref, alpha_ref, o_ref):
    z = x_ref[...] + agg_ref[...]
    h = jnp.maximum(jnp.dot(z, w1_ref[...], preferred_element_type=jnp.float32)
                    + b1_ref[...], 0.0)
    xn = jnp.dot(h, w2_ref[...], preferred_element_type=jnp.float32) + b2_ref[...]
    sk = jnp.dot(x_ref[...], skw_ref[...], preferred_element_type=jnp.float32) + skb_ref[...]
    a = alpha_ref[0, 0]
    o_ref[...] = a * sk + (1.0 - a) * xn


def _mlp0(x, agg, w1, b1, w2, b2, skw, skb, alpha):
    return pl.pallas_call(
        _mlp0_body,
        grid=(N // BN,),
        in_specs=[
            pl.BlockSpec((BN, D_IN), lambda i: (i, 0)),
            pl.BlockSpec((BN, D_IN), lambda i: (i, 0)),
            pl.BlockSpec((D_IN, D_H), lambda i: (0, 0)),
            pl.BlockSpec((1, D_H), lambda i: (0, 0)),
            pl.BlockSpec((D_H, D_H), lambda i: (0, 0)),
            pl.BlockSpec((1, D_H), lambda i: (0, 0)),
            pl.BlockSpec((D_IN, D_H), lambda i: (0, 0)),
            pl.BlockSpec((1, D_H), lambda i: (0, 0)),
            pl.BlockSpec(memory_space=pltpu.SMEM),
        ],
        out_specs=pl.BlockSpec((BN, D_H), lambda i: (i, 0)),
        out_shape=jax.ShapeDtypeStruct((N, D_H), jnp.float32),
    )(x, agg, w1, b1.reshape(1, D_H), w2, b2.reshape(1, D_H),
      skw, skb.reshape(1, D_H), alpha.reshape(1, 1))


def _mlp_body(x_ref, agg_ref, w1_ref, b1_ref, w2_ref, b2_ref, alpha_ref, o_ref):
    x = x_ref[...]
    z = x + agg_ref[...]
    h = jnp.maximum(jnp.dot(z, w1_ref[...], preferred_element_type=jnp.float32)
                    + b1_ref[...], 0.0)
    xn = jnp.dot(h, w2_ref[...], preferred_element_type=jnp.float32) + b2_ref[...]
    a = alpha_ref[0, 0]
    o_ref[...] = a * x + (1.0 - a) * xn


def _mlp(x, agg, w1, b1, w2, b2, alpha):
    return pl.pallas_call(
        _mlp_body,
        grid=(N // BN,),
        in_specs=[
            pl.BlockSpec((BN, D_H), lambda i: (i, 0)),
            pl.BlockSpec((BN, D_H), lambda i: (i, 0)),
            pl.BlockSpec((D_H, D_H), lambda i: (0, 0)),
            pl.BlockSpec((1, D_H), lambda i: (0, 0)),
            pl.BlockSpec((D_H, D_H), lambda i: (0, 0)),
            pl.BlockSpec((1, D_H), lambda i: (0, 0)),
            pl.BlockSpec(memory_space=pltpu.SMEM),
        ],
        out_specs=pl.BlockSpec((BN, D_H), lambda i: (i, 0)),
        out_shape=jax.ShapeDtypeStruct((N, D_H), jnp.float32),
    )(x, agg, w1, b1.reshape(1, D_H), w2, b2.reshape(1, D_H), alpha.reshape(1, 1))


def _aggregate(x, src, dst, e_flat, S):
    e = jnp.concatenate([lax.slice(e_flat, (p * E_PAD, 0), (p * E_PAD + E, 128))
                         for p in range(S)], axis=1)
    m = jax.nn.relu(x[src] + e)
    return jax.ops.segment_sum(m, dst, num_segments=N)


def kernel(x, edge_index, edge_attr,
           lin_w0, lin_b0, w1_0, b1_0, w2_0, b2_0, alpha0,
           lin_w1, lin_b1, w1_1, b1_1, w2_1, b2_1, alpha1,
           lin_w2, lin_b2, w1_2, b1_2, w2_2, b2_2, alpha2,
           skip_w0, skip_b0):
    src = edge_index[0]
    dst = edge_index[1]
    ea_pad = jnp.pad(edge_attr, ((0, E_PAD - E), (0, 0)))

    e0 = _edge_mlp(ea_pad, lin_w0, lin_b0, 2)
    e1 = _edge_mlp(ea_pad, lin_w1, lin_b1, 4)
    e2 = _edge_mlp(ea_pad, lin_w2, lin_b2, 4)

    agg0 = _aggregate(x, src, dst, e0, 2)
    x1 = _mlp0(x, agg0, w1_0, b1_0, w2_0, b2_0, skip_w0, skip_b0, alpha0)
    agg1 = _aggregate(x1, src, dst, e1, 4)
    x2 = _mlp(x1, agg1, w1_1, b1_1, w2_1, b2_1, alpha1)
    agg2 = _aggregate(x2, src, dst, e2, 4)
    x3 = _mlp(x2, agg2, w1_2, b1_2, w2_2, b2_2, alpha2)

    return jnp.concatenate([x, x1, x2, x3], axis=-1)


# speedup vs baseline: 1.3034x; 1.3034x over previous
import functools
import jax
import jax.numpy as jnp
from jax import lax
from jax.experimental import pallas as pl
from jax.experimental.pallas import tpu as pltpu
from jax.experimental.pallas import tpu_sc as plsc

N = 10000
E = 160000
D_IN = 256
D_H = 512
D_E = 16

NTILES = 16
CHUNK = 128
E_PAD = 163840
CH_PER_TILE = E_PAD // (NTILES * CHUNK)
IDX_ROWS = E_PAD // CHUNK
ROWS_PER_TILE = 632
N_PAD = NTILES * ROWS_PER_TILE
ACC_ROWS = N_PAD

BE = 2048
BN = 1000


def _edge_mlp_body(a_ref, lw_ref, lb_ref, o_ref):
    o_ref[...] = (jnp.dot(a_ref[...], lw_ref[...],
                          preferred_element_type=jnp.float32) + lb_ref[...])


def _edge_mlp(edge_attr_pad, lw, lb, S):
    nblk = E_PAD // BE
    return pl.pallas_call(
        _edge_mlp_body,
        grid=(S, nblk),
        in_specs=[
            pl.BlockSpec((BE, D_E), lambda p, i: (i, 0)),
            pl.BlockSpec((D_E, 128), lambda p, i: (0, p)),
            pl.BlockSpec((1, 128), lambda p, i: (0, p)),
        ],
        out_specs=pl.BlockSpec((BE, 128), lambda p, i: (p * nblk + i, 0)),
        out_shape=jax.ShapeDtypeStruct((S * E_PAD, 128), jnp.float32),
    )(edge_attr_pad, lw, lb.reshape(1, S * 128))


def _make_sc_agg(S):
    spc = S // 2
    mesh = plsc.VectorSubcoreMesh(core_axis_name="c", subcore_axis_name="s",
                                  num_cores=2, num_subcores=NTILES)

    @functools.partial(
        pl.kernel,
        out_type=jax.ShapeDtypeStruct((S * N_PAD, 128), jnp.float32),
        mesh=mesh,
        scratch_types=[
            pltpu.VMEM((16, CHUNK), jnp.int32),
            pltpu.VMEM((16, CHUNK), jnp.int32),
            pltpu.VMEM((CHUNK, 128), jnp.float32),
            pltpu.VMEM((CHUNK, 128), jnp.float32),
            pltpu.VMEM_SHARED((ACC_ROWS, 128), jnp.float32),
            pltpu.SemaphoreType.DMA,
            pltpu.SemaphoreType.DMA,
        ],
    )
    def sc_agg(x_hbm, e_hbm, srcp_hbm, dst_hbm, zeros_hbm, out_hbm,
               src_v, dst_v, g_v, e_v, acc, sem_g, sem_e):
        c = lax.axis_index("c")
        s = lax.axis_index("s")
        row0 = s * CH_PER_TILE
        for q in range(spc):
            p = c + 2 * q
            pltpu.sync_copy(zeros_hbm, acc.at[pl.ds(s * ROWS_PER_TILE, ROWS_PER_TILE)])
            plsc.subcore_barrier()

            def group(grp, carry):
                g0 = row0 + grp * 16
                pltpu.sync_copy(srcp_hbm.at[pl.ds(p * IDX_ROWS + g0, 16)], src_v)
                pltpu.sync_copy(dst_hbm.at[pl.ds(g0, 16)], dst_v)

                def step(j, carry1):
                    ebase = p * E_PAD + (g0 + j) * CHUNK
                    cp_e = pltpu.async_copy(e_hbm.at[pl.ds(ebase, CHUNK)], e_v, sem_e)
                    cp_g = pltpu.async_copy(x_hbm.at[src_v.at[j]], g_v, sem_g)
                    cp_e.wait()
                    cp_g.wait()

                    def relu_row(r, carry2):
                        for k in range(8):
                            sl = pl.ds(k * 16, 16)
                            g_v[r, sl] = jnp.maximum(g_v[r, sl] + e_v[r, sl], 0.0)
                        return carry2

                    lax.fori_loop(0, CHUNK, relu_row, 0, unroll=2)
                    pltpu.sync_copy(g_v, acc.at[dst_v.at[j]], add=True)
                    return carry1

                lax.fori_loop(0, 16, step, 0)
                return carry

            lax.fori_loop(0, CH_PER_TILE // 16, group, 0)
            plsc.subcore_barrier()
            pltpu.sync_copy(acc.at[pl.ds(s * ROWS_PER_TILE, ROWS_PER_TILE)],
                            out_hbm.at[pl.ds(p * N_PAD + s * ROWS_PER_TILE,
                                             ROWS_PER_TILE)])

    return sc_agg


def _aggregate(x_flat, e_flat, srcp, dst2, zeros, S):
    return _make_sc_agg(S)(x_flat, e_flat, srcp, dst2, zeros)


def _mlp0_body(x_ref, agg_ref, w1_ref, b1_ref, w2_ref, b2_ref,
               skw_ref, skb_ref, alpha_ref, oslab_ref, ofull_ref):
    x = x_ref[...]
    agg = jnp.concatenate([agg_ref[p] for p in range(2)], axis=-1)
    z = x + agg
    h = jnp.maximum(jnp.dot(z, w1_ref[...], preferred_element_type=jnp.float32)
                    + b1_ref[...], 0.0)
    xn = jnp.dot(h, w2_ref[...], preferred_element_type=jnp.float32) + b2_ref[...]
    sk = jnp.dot(x, skw_ref[...], preferred_element_type=jnp.float32) + skb_ref[...]
    a = alpha_ref[0, 0]
    res = a * sk + (1.0 - a) * xn
    ofull_ref[...] = res
    for p in range(4):
        oslab_ref[p] = res[:, p * 128:(p + 1) * 128]


def _mlp0(x, agg, w1, b1, w2, b2, skw, skb, alpha):
    return pl.pallas_call(
        _mlp0_body,
        grid=(N // BN,),
        in_specs=[
            pl.BlockSpec((BN, D_IN), lambda i: (i, 0)),
            pl.BlockSpec((2, BN, 128), lambda i: (0, i, 0)),
            pl.BlockSpec((D_IN, D_H), lambda i: (0, 0)),
            pl.BlockSpec((1, D_H), lambda i: (0, 0)),
            pl.BlockSpec((D_H, D_H), lambda i: (0, 0)),
            pl.BlockSpec((1, D_H), lambda i: (0, 0)),
            pl.BlockSpec((D_IN, D_H), lambda i: (0, 0)),
            pl.BlockSpec((1, D_H), lambda i: (0, 0)),
            pl.BlockSpec(memory_space=pltpu.SMEM),
        ],
        out_specs=(
            pl.BlockSpec((4, BN, 128), lambda i: (0, i, 0)),
            pl.BlockSpec((BN, D_H), lambda i: (i, 0)),
        ),
        out_shape=(
            jax.ShapeDtypeStruct((4, N, 128), jnp.float32),
            jax.ShapeDtypeStruct((N, D_H), jnp.float32),
        ),
    )(x, agg, w1, b1.reshape(1, D_H), w2, b2.reshape(1, D_H),
      skw, skb.reshape(1, D_H), alpha.reshape(1, 1))


def _make_mlp_body(want_slab):
    def body(x_ref, agg_ref, w1_ref, b1_ref, w2_ref, b2_ref, alpha_ref, *outs):
        x = jnp.concatenate([x_ref[p] for p in range(4)], axis=-1)
        agg = jnp.concatenate([agg_ref[p] for p in range(4)], axis=-1)
        z = x + agg
        h = jnp.maximum(jnp.dot(z, w1_ref[...], preferred_element_type=jnp.float32)
                        + b1_ref[...], 0.0)
        xn = jnp.dot(h, w2_ref[...], preferred_element_type=jnp.float32) + b2_ref[...]
        a = alpha_ref[0, 0]
        res = a * x + (1.0 - a) * xn
        if want_slab:
            oslab_ref, ofull_ref = outs
            for p in range(4):
                oslab_ref[p] = res[:, p * 128:(p + 1) * 128]
        else:
            (ofull_ref,) = outs
        ofull_ref[...] = res
    return body


def _mlp(x_slab, agg, w1, b1, w2, b2, alpha, want_slab):
    out_specs = [pl.BlockSpec((BN, D_H), lambda i: (i, 0))]
    out_shape = [jax.ShapeDtypeStruct((N, D_H), jnp.float32)]
    if want_slab:
        out_specs.insert(0, pl.BlockSpec((4, BN, 128), lambda i: (0, i, 0)))
        out_shape.insert(0, jax.ShapeDtypeStruct((4, N, 128), jnp.float32))
    return pl.pallas_call(
        _make_mlp_body(want_slab),
        grid=(N // BN,),
        in_specs=[
            pl.BlockSpec((4, BN, 128), lambda i: (0, i, 0)),
            pl.BlockSpec((4, BN, 128), lambda i: (0, i, 0)),
            pl.BlockSpec((D_H, D_H), lambda i: (0, 0)),
            pl.BlockSpec((1, D_H), lambda i: (0, 0)),
            pl.BlockSpec((D_H, D_H), lambda i: (0, 0)),
            pl.BlockSpec((1, D_H), lambda i: (0, 0)),
            pl.BlockSpec(memory_space=pltpu.SMEM),
        ],
        out_specs=tuple(out_specs),
        out_shape=tuple(out_shape),
    )(x_slab, agg, w1, b1.reshape(1, D_H), w2, b2.reshape(1, D_H),
      alpha.reshape(1, 1))


def kernel(x, edge_index, edge_attr,
           lin_w0, lin_b0, w1_0, b1_0, w2_0, b2_0, alpha0,
           lin_w1, lin_b1, w1_1, b1_1, w2_1, b2_1, alpha1,
           lin_w2, lin_b2, w1_2, b1_2, w2_2, b2_2, alpha2,
           skip_w0, skip_b0):
    src = edge_index[0]
    dst = edge_index[1]
    ea_pad = jnp.pad(edge_attr, ((0, E_PAD - E), (0, 0)))

    def srcp_table(S):
        sp = src[None, :] + (jnp.arange(S, dtype=jnp.int32) * N)[:, None]
        sp = jnp.pad(sp, ((0, 0), (0, E_PAD - E)))
        return sp.reshape(S * IDX_ROWS, CHUNK)

    srcp2 = srcp_table(2)
    srcp4 = srcp_table(4)
    dst2 = jnp.pad(dst, (0, E_PAD - E), constant_values=N).reshape(IDX_ROWS, CHUNK)
    zeros = jnp.zeros((ROWS_PER_TILE, 128), jnp.float32)

    e0 = _edge_mlp(ea_pad, lin_w0, lin_b0, 2)
    e1 = _edge_mlp(ea_pad, lin_w1, lin_b1, 4)
    e2 = _edge_mlp(ea_pad, lin_w2, lin_b2, 4)

    x_flat = x.reshape(N, 2, 128).transpose(1, 0, 2).reshape(2 * N, 128)

    agg0 = _aggregate(x_flat, e0, srcp2, dst2, zeros, 2).reshape(2, N_PAD, 128)
    x1_slab, x1 = _mlp0(x, agg0, w1_0, b1_0, w2_0, b2_0, skip_w0, skip_b0, alpha0)

    agg1 = _aggregate(x1_slab.reshape(4 * N, 128), e1, srcp4, dst2, zeros, 4
                      ).reshape(4, N_PAD, 128)
    x2_slab, x2 = _mlp(x1_slab, agg1, w1_1, b1_1, w2_1, b2_1, alpha1, True)

    agg2 = _aggregate(x2_slab.reshape(4 * N, 128), e2, srcp4, dst2, zeros, 4
                      ).reshape(4, N_PAD, 128)
    x3 = _mlp(x2_slab, agg2, w1_2, b1_2, w2_2, b2_2, alpha2, False)[0]

    return jnp.concatenate([x, x1, x2, x3], axis=-1)


# speedup vs baseline: 1.3275x; 1.0185x over previous
import functools
import jax
import jax.numpy as jnp
from jax import lax
from jax.experimental import pallas as pl
from jax.experimental.pallas import tpu as pltpu
from jax.experimental.pallas import tpu_sc as plsc

N = 10000
E = 160000
D_IN = 256
D_H = 512
D_E = 16

NTILES = 16
CHUNK = 64
CH_PER_TILE = 162
E_PAD = NTILES * CH_PER_TILE * CHUNK
T_ITERS = CH_PER_TILE + 3
ROWS_PER_TILE = 632
ACC_ROWS = N + 8

BE = 2048
BN = 1000


def _edge_mlp_body(a_ref, lw_ref, lb_ref, o_ref):
    o_ref[...] = (jnp.dot(a_ref[...], lw_ref[...],
                          preferred_element_type=jnp.float32) + lb_ref[...])


def _edge_mlp(edge_attr_pad, lw, lb, S):
    nblk = E_PAD // BE
    return pl.pallas_call(
        _edge_mlp_body,
        grid=(S, nblk),
        in_specs=[
            pl.BlockSpec((BE, D_E), lambda p, i: (i, 0)),
            pl.BlockSpec((D_E, 128), lambda p, i: (0, p)),
            pl.BlockSpec((1, 128), lambda p, i: (0, p)),
        ],
        out_specs=pl.BlockSpec((BE, 128), lambda p, i: (p * nblk + i, 0)),
        out_shape=jax.ShapeDtypeStruct((S * E_PAD, 128), jnp.float32),
    )(edge_attr_pad, lw, lb.reshape(1, S * 128))


def _make_sc_agg(S):
    spc = S // 2
    mesh = plsc.VectorSubcoreMesh(core_axis_name="c", subcore_axis_name="s",
                                  num_cores=2, num_subcores=NTILES)

    @functools.partial(
        pl.kernel,
        out_type=jax.ShapeDtypeStruct((S * N, 128), jnp.float32),
        mesh=mesh,
        scratch_types=[
            [pltpu.VMEM((CHUNK,), jnp.int32) for _ in range(3)],
            [pltpu.VMEM((CHUNK,), jnp.int32) for _ in range(3)],
            [pltpu.VMEM((CHUNK, 128), jnp.float32) for _ in range(3)],
            [pltpu.VMEM((CHUNK, 128), jnp.float32) for _ in range(3)],
            pltpu.VMEM_SHARED((ACC_ROWS, 128), jnp.float32),
            [pltpu.SemaphoreType.DMA for _ in range(3)],
            [pltpu.SemaphoreType.DMA for _ in range(3)],
            [pltpu.SemaphoreType.DMA for _ in range(3)],
        ],
    )
    def sc_agg(x_hbm, e_hbm, srcp_hbm, dst_hbm, zeros_hbm, out_hbm,
               si, di, gb, eb, acc, sem_i, sem_d, sem_s):
        c = lax.axis_index("c")
        s = lax.axis_index("s")
        ch0 = s * CH_PER_TILE

        def idx_copies(slot, ch, p):
            off = ch * CHUNK
            return (pltpu.make_async_copy(srcp_hbm.at[pl.ds(p * E_PAD + off, CHUNK)],
                                          si[slot], sem_i[slot]),
                    pltpu.make_async_copy(dst_hbm.at[pl.ds(off, CHUNK)],
                                          di[slot], sem_i[slot]))

        def data_copies(slot, ch, p):
            return (pltpu.make_async_copy(e_hbm.at[pl.ds(p * E_PAD + ch * CHUNK, CHUNK)],
                                          eb[slot], sem_d[slot]),
                    pltpu.make_async_copy(x_hbm.at[si[slot]], gb[slot], sem_d[slot]))

        def scatter_copy(slot):
            return pltpu.make_async_copy(gb[slot], acc.at[di[slot]], sem_s[slot])

        for q in range(spc):
            p = c + 2 * q
            @pl.when(s < 15)
            def _():
                pltpu.sync_copy(zeros_hbm,
                                acc.at[pl.ds(s * ROWS_PER_TILE, ROWS_PER_TILE)])

            @pl.when(s == 15)
            def _():
                pltpu.sync_copy(zeros_hbm.at[pl.ds(0, 528)],
                                acc.at[pl.ds(15 * ROWS_PER_TILE, 528)])

            plsc.subcore_barrier()

            @pl.loop(0, T_ITERS, step=3)
            def _(outer):
                for b in range(3):
                    j = outer + b
                    slot_a = b
                    slot_g = (b + 2) % 3
                    slot_c = (b + 1) % 3

                    @pl.when(j >= 3)
                    def _():
                        scatter_copy(slot_a).wait()

                    @pl.when(j < CH_PER_TILE)
                    def _():
                        for cp in idx_copies(slot_a, ch0 + j, p):
                            cp.start()

                    @pl.when((j >= 1) & (j < CH_PER_TILE + 1))
                    def _():
                        for cp in idx_copies(slot_g, ch0 + j - 1, p):
                            cp.wait()
                        for cp in data_copies(slot_g, ch0 + j - 1, p):
                            cp.start()

                    @pl.when((j >= 2) & (j < CH_PER_TILE + 2))
                    def _():
                        for cp in data_copies(slot_c, ch0 + j - 2, p):
                            cp.wait()

                        def relu_row(r, carry2):
                            for k in range(8):
                                sl = pl.ds(k * 16, 16)
                                gb[slot_c][r, sl] = jnp.maximum(
                                    gb[slot_c][r, sl] + eb[slot_c][r, sl], 0.0)
                            return carry2

                        lax.fori_loop(0, CHUNK, relu_row, 0, unroll=2)
                        scatter_copy(slot_c).start(add=True)

            plsc.subcore_barrier()
            @pl.when(s < 15)
            def _():
                pltpu.sync_copy(acc.at[pl.ds(s * ROWS_PER_TILE, ROWS_PER_TILE)],
                                out_hbm.at[pl.ds(p * N + s * ROWS_PER_TILE,
                                                 ROWS_PER_TILE)])

            @pl.when(s == 15)
            def _():
                pltpu.sync_copy(acc.at[pl.ds(15 * ROWS_PER_TILE, 520)],
                                out_hbm.at[pl.ds(p * N + 15 * ROWS_PER_TILE, 520)])

    return sc_agg


def _aggregate(x_flat, e_flat, srcp, dst2, zeros, S):
    return _make_sc_agg(S)(x_flat, e_flat, srcp, dst2, zeros)


def _mlp0_body(x_ref, agg_ref, w1_ref, b1_ref, w2_ref, b2_ref,
               skw_ref, skb_ref, alpha_ref, oslab_ref, ofull_ref):
    x = x_ref[...]
    agg = jnp.concatenate([agg_ref[p] for p in range(2)], axis=-1)
    z = x + agg
    h = jnp.maximum(jnp.dot(z, w1_ref[...], preferred_element_type=jnp.float32)
                    + b1_ref[...], 0.0)
    xn = jnp.dot(h, w2_ref[...], preferred_element_type=jnp.float32) + b2_ref[...]
    sk = jnp.dot(x, skw_ref[...], preferred_element_type=jnp.float32) + skb_ref[...]
    a = alpha_ref[0, 0]
    res = a * sk + (1.0 - a) * xn
    ofull_ref[...] = res
    for p in range(4):
        oslab_ref[p] = res[:, p * 128:(p + 1) * 128]


def _mlp0(x, agg, w1, b1, w2, b2, skw, skb, alpha):
    return pl.pallas_call(
        _mlp0_body,
        grid=(N // BN,),
        in_specs=[
            pl.BlockSpec((BN, D_IN), lambda i: (i, 0)),
            pl.BlockSpec((2, BN, 128), lambda i: (0, i, 0)),
            pl.BlockSpec((D_IN, D_H), lambda i: (0, 0)),
            pl.BlockSpec((1, D_H), lambda i: (0, 0)),
            pl.BlockSpec((D_H, D_H), lambda i: (0, 0)),
            pl.BlockSpec((1, D_H), lambda i: (0, 0)),
            pl.BlockSpec((D_IN, D_H), lambda i: (0, 0)),
            pl.BlockSpec((1, D_H), lambda i: (0, 0)),
            pl.BlockSpec(memory_space=pltpu.SMEM),
        ],
        out_specs=(
            pl.BlockSpec((4, BN, 128), lambda i: (0, i, 0)),
            pl.BlockSpec((BN, D_H), lambda i: (i, 0)),
        ),
        out_shape=(
            jax.ShapeDtypeStruct((4, N, 128), jnp.float32),
            jax.ShapeDtypeStruct((N, D_H), jnp.float32),
        ),
    )(x, agg, w1, b1.reshape(1, D_H), w2, b2.reshape(1, D_H),
      skw, skb.reshape(1, D_H), alpha.reshape(1, 1))


def _make_mlp_body(want_slab):
    def body(x_ref, agg_ref, w1_ref, b1_ref, w2_ref, b2_ref, alpha_ref, *outs):
        x = jnp.concatenate([x_ref[p] for p in range(4)], axis=-1)
        agg = jnp.concatenate([agg_ref[p] for p in range(4)], axis=-1)
        z = x + agg
        h = jnp.maximum(jnp.dot(z, w1_ref[...], preferred_element_type=jnp.float32)
                        + b1_ref[...], 0.0)
        xn = jnp.dot(h, w2_ref[...], preferred_element_type=jnp.float32) + b2_ref[...]
        a = alpha_ref[0, 0]
        res = a * x + (1.0 - a) * xn
        if want_slab:
            oslab_ref, ofull_ref = outs
            for p in range(4):
                oslab_ref[p] = res[:, p * 128:(p + 1) * 128]
        else:
            (ofull_ref,) = outs
        ofull_ref[...] = res
    return body


def _mlp(x_slab, agg, w1, b1, w2, b2, alpha, want_slab):
    out_specs = [pl.BlockSpec((BN, D_H), lambda i: (i, 0))]
    out_shape = [jax.ShapeDtypeStruct((N, D_H), jnp.float32)]
    if want_slab:
        out_specs.insert(0, pl.BlockSpec((4, BN, 128), lambda i: (0, i, 0)))
        out_shape.insert(0, jax.ShapeDtypeStruct((4, N, 128), jnp.float32))
    return pl.pallas_call(
        _make_mlp_body(want_slab),
        grid=(N // BN,),
        in_specs=[
            pl.BlockSpec((4, BN, 128), lambda i: (0, i, 0)),
            pl.BlockSpec((4, BN, 128), lambda i: (0, i, 0)),
            pl.BlockSpec((D_H, D_H), lambda i: (0, 0)),
            pl.BlockSpec((1, D_H), lambda i: (0, 0)),
            pl.BlockSpec((D_H, D_H), lambda i: (0, 0)),
            pl.BlockSpec((1, D_H), lambda i: (0, 0)),
            pl.BlockSpec(memory_space=pltpu.SMEM),
        ],
        out_specs=tuple(out_specs),
        out_shape=tuple(out_shape),
    )(x_slab, agg, w1, b1.reshape(1, D_H), w2, b2.reshape(1, D_H),
      alpha.reshape(1, 1))


def kernel(x, edge_index, edge_attr,
           lin_w0, lin_b0, w1_0, b1_0, w2_0, b2_0, alpha0,
           lin_w1, lin_b1, w1_1, b1_1, w2_1, b2_1, alpha1,
           lin_w2, lin_b2, w1_2, b1_2, w2_2, b2_2, alpha2,
           skip_w0, skip_b0):
    src = edge_index[0]
    dst = edge_index[1]
    ea_pad = jnp.pad(edge_attr, ((0, E_PAD - E), (0, 0)))

    def srcp_table(S):
        sp = src[None, :] + (jnp.arange(S, dtype=jnp.int32) * N)[:, None]
        sp = jnp.pad(sp, ((0, 0), (0, E_PAD - E)))
        return sp.reshape(S * E_PAD)

    srcp2 = srcp_table(2)
    srcp4 = srcp_table(4)
    dst2 = jnp.pad(dst, (0, E_PAD - E), constant_values=N)
    zeros = jnp.zeros((ROWS_PER_TILE, 128), jnp.float32)

    e0 = _edge_mlp(ea_pad, lin_w0, lin_b0, 2)
    e1 = _edge_mlp(ea_pad, lin_w1, lin_b1, 4)
    e2 = _edge_mlp(ea_pad, lin_w2, lin_b2, 4)

    x_flat = x.reshape(N, 2, 128).transpose(1, 0, 2).reshape(2 * N, 128)

    agg0 = _aggregate(x_flat, e0, srcp2, dst2, zeros, 2).reshape(2, N, 128)
    x1_slab, x1 = _mlp0(x, agg0, w1_0, b1_0, w2_0, b2_0, skip_w0, skip_b0, alpha0)

    agg1 = _aggregate(x1_slab.reshape(4 * N, 128), e1, srcp4, dst2, zeros, 4
                      ).reshape(4, N, 128)
    x2_slab, x2 = _mlp(x1_slab, agg1, w1_1, b1_1, w2_1, b2_1, alpha1, True)

    agg2 = _aggregate(x2_slab.reshape(4 * N, 128), e2, srcp4, dst2, zeros, 4
                      ).reshape(4, N, 128)
    x3 = _mlp(x2_slab, agg2, w1_2, b1_2, w2_2, b2_2, alpha2, False)[0]

    return jnp.concatenate([x, x1, x2, x3], axis=-1)
